# baseline (speedup 1.0000x reference)
import functools

import jax
import jax.numpy as jnp
from jax import lax
from jax.experimental import pallas as pl
from jax.experimental.pallas import tpu as pltpu
from jax.experimental.pallas import tpu_sc as plsc
from jax.experimental.layout import Layout, with_layout_constraint

LATENT = 64
_NC = 2
_NS = 16
_NW = _NC * _NS
_CHUNK = 128


def _sc_gather(idx_all, table):
    n = idx_all.shape[0]
    bpw = n // _NW
    nchunk = bpw // _CHUNK

    mesh = plsc.VectorSubcoreMesh(core_axis_name="c", subcore_axis_name="s")

    @functools.partial(
        pl.kernel,
        out_type=jax.ShapeDtypeStruct((n, 2 * LATENT), jnp.float32),
        mesh=mesh,
        scratch_types=[
            pltpu.VMEM((bpw,), jnp.int32),
            pltpu.VMEM((2, _CHUNK, 2 * LATENT), jnp.float32),
            pltpu.SemaphoreType.DMA,
        ],
    )
    def gather_kernel(idx_hbm, table_hbm, out_hbm, idx_v, rows_v, sem):
        wid = lax.axis_index("s") * _NC + lax.axis_index("c")
        base = wid * bpw
        pltpu.sync_copy(idx_hbm.at[pl.ds(base, bpw)], idx_v)

        def fire(c):
            return pltpu.make_async_copy(
                table_hbm.at[idx_v.at[pl.ds(c * _CHUNK, _CHUNK)]],
                rows_v.at[c % 2], sem)

        prev = fire(0)
        prev.start()
        for c in range(nchunk):
            if c + 1 < nchunk:
                nxt = fire(c + 1)
                nxt.start()
            prev.wait()
            pltpu.sync_copy(
                rows_v.at[c % 2],
                out_hbm.at[pl.ds(base + c * _CHUNK, _CHUNK)])
            if c + 1 < nchunk:
                prev = nxt

    return gather_kernel(idx_all, table)


def _mlp_body(s_ref, t_ref, ps_ref, pt_ref, w1_ref, b1_ref, w2_ref, b2_ref,
              o_ref):
    sp = s_ref[...]
    tp = t_ref[...]
    ps = ps_ref[...]
    pt = pt_ref[...]
    s_emb = sp[:, :LATENT] * (1.0 - ps) + sp[:, LATENT:] * ps
    t_emb = tp[:, :LATENT] * (1.0 - pt) + tp[:, LATENT:] * pt
    e = s_emb * t_emb
    h = jnp.dot(e, w1_ref[...], preferred_element_type=jnp.float32)
    h = jnp.maximum(h + b1_ref[...], 0.0)
    z = jnp.sum(h * w2_ref[...], axis=1, keepdims=True) + b2_ref[...]
    o_ref[...] = jax.nn.sigmoid(z)


def kernel(src, tgt, table, W1, b1, W2, b2):
    B = src.shape[0]
    t_row = with_layout_constraint(
        table, Layout(major_to_minor=(0, 1), tiling=()))
    t2 = t_row.reshape(table.shape[0] // 2, 2 * LATENT)
    idx_all = jnp.concatenate([src, tgt], axis=0)
    emb = _sc_gather(lax.shift_right_logical(idx_all, 1), t2)
    ps = lax.bitwise_and(src, 1).astype(jnp.float32).reshape(B, 1)
    pt = lax.bitwise_and(tgt, 1).astype(jnp.float32).reshape(B, 1)

    blk = 2048
    nblk = B // blk
    out = pl.pallas_call(
        _mlp_body,
        grid=(nblk,),
        in_specs=[
            pl.BlockSpec((blk, 2 * LATENT), lambda i: (i, 0)),
            pl.BlockSpec((blk, 2 * LATENT), lambda i, _n=nblk: (i + _n, 0)),
            pl.BlockSpec((blk, 1), lambda i: (i, 0)),
            pl.BlockSpec((blk, 1), lambda i: (i, 0)),
            pl.BlockSpec((LATENT, LATENT), lambda i: (0, 0)),
            pl.BlockSpec((1, LATENT), lambda i: (0, 0)),
            pl.BlockSpec((1, LATENT), lambda i: (0, 0)),
            pl.BlockSpec((1, 1), lambda i: (0, 0)),
        ],
        out_specs=pl.BlockSpec((blk, 1), lambda i: (i, 0)),
        out_shape=jax.ShapeDtypeStruct((B, 1), jnp.float32),
    )(emb, emb, ps, pt, W1, b1.reshape(1, LATENT), W2.reshape(1, LATENT),
      b2.reshape(1, 1))
    return out


# speedup vs baseline: 1.6882x; 1.6882x over previous
import functools

import jax
import jax.numpy as jnp
from jax import lax
from jax.experimental import pallas as pl
from jax.experimental.pallas import tpu as pltpu
from jax.experimental.pallas import tpu_sc as plsc
from jax.experimental.layout import Layout, with_layout_constraint

LATENT = 64
_NC = 2
_NS = 16
_NW = _NC * _NS
_CHUNK = 128


def _sc_gather(idx_all, table):
    n = idx_all.shape[0]
    bpw = n // _NW
    nchunk = bpw // _CHUNK

    mesh = plsc.VectorSubcoreMesh(core_axis_name="c", subcore_axis_name="s")

    @functools.partial(
        pl.kernel,
        out_type=jax.ShapeDtypeStruct((n, LATENT), jnp.float32),
        mesh=mesh,
        scratch_types=[
            pltpu.VMEM((bpw,), jnp.int32),
            pltpu.VMEM((2, _CHUNK, LATENT), jnp.float32),
            pltpu.SemaphoreType.DMA,
        ],
        compiler_params=pltpu.CompilerParams(use_tc_tiling_on_sc=False),
    )
    def gather_kernel(idx_hbm, table_hbm, out_hbm, idx_v, rows_v, sem):
        wid = lax.axis_index("s") * _NC + lax.axis_index("c")
        base = wid * bpw
        pltpu.sync_copy(idx_hbm.at[pl.ds(base, bpw)], idx_v)

        def fire(c):
            return pltpu.make_async_copy(
                table_hbm.at[idx_v.at[pl.ds(c * _CHUNK, _CHUNK)]],
                rows_v.at[c % 2], sem)

        prev = fire(0)
        prev.start()
        for c in range(nchunk):
            if c + 1 < nchunk:
                nxt = fire(c + 1)
                nxt.start()
            prev.wait()
            pltpu.sync_copy(
                rows_v.at[c % 2],
                out_hbm.at[pl.ds(base + c * _CHUNK, _CHUNK)])
            if c + 1 < nchunk:
                prev = nxt

    return gather_kernel(idx_all, table)


def _mlp_body(s_ref, t_ref, w1_ref, b1_ref, w2_ref, b2_ref, o_ref):
    e = s_ref[...] * t_ref[...]
    h = jnp.dot(e, w1_ref[...], preferred_element_type=jnp.float32)
    h = jnp.maximum(h + b1_ref[...], 0.0)
    z = jnp.sum(h * w2_ref[...], axis=1, keepdims=True) + b2_ref[...]
    o_ref[...] = jax.nn.sigmoid(z)


def kernel(src, tgt, table, W1, b1, W2, b2):
    B = src.shape[0]
    t_row = with_layout_constraint(
        table, Layout(major_to_minor=(0, 1), tiling=((1, 128),)))
    idx_all = jnp.concatenate([src, tgt], axis=0)
    emb = _sc_gather(idx_all, t_row)

    blk = 2048
    nblk = B // blk
    out = pl.pallas_call(
        _mlp_body,
        grid=(nblk,),
        in_specs=[
            pl.BlockSpec((blk, LATENT), lambda i: (i, 0)),
            pl.BlockSpec((blk, LATENT), lambda i, _n=nblk: (i + _n, 0)),
            pl.BlockSpec((LATENT, LATENT), lambda i: (0, 0)),
            pl.BlockSpec((1, LATENT), lambda i: (0, 0)),
            pl.BlockSpec((1, LATENT), lambda i: (0, 0)),
            pl.BlockSpec((1, 1), lambda i: (0, 0)),
        ],
        out_specs=pl.BlockSpec((blk, 1), lambda i: (i, 0)),
        out_shape=jax.ShapeDtypeStruct((B, 1), jnp.float32),
    )(emb, emb, W1, b1.reshape(1, LATENT), W2.reshape(1, LATENT),
      b2.reshape(1, 1))
    return out
